# Initial kernel scaffold; baseline (speedup 1.0000x reference)
import jax
import jax.numpy as jnp
from jax.experimental import pallas as pl


def kernel(species, coordinates, net_charge, params):
    raise NotImplementedError("write your pallas kernel here")


# speedup vs baseline: 1.0467x; 1.0467x over previous
import jax
import jax.numpy as jnp
from jax.experimental import pallas as pl
from jax.experimental.pallas import tpu as pltpu

A0 = 0.529177249
SIG2 = [0.5515909**2, 1.8886297**2, 1.3225029**2, 1.2316629**2,
        2.1884933**2, 1.7750372**2, 1.3677907**2, 1.3820058**2]
NM, NA, NS = 128, 64, 8
G = 4
T = G * NA
STEPS = NM // G


def _celu(x):
    return jnp.where(x > 0, x, 0.1 * (jnp.exp(jnp.minimum(x * 10.0, 0.0)) - 1.0))


def _softplus(x):
    return jnp.maximum(x, 0.0) + jnp.log(1.0 + jnp.exp(-jnp.abs(x)))


def _erf(x):
    t = 1.0 / (1.0 + 0.3275911 * x)
    p = t * (0.254829592 + t * (-0.284496736 + t * (1.421413741
              + t * (-1.453152027 + t * 1.061405429))))
    return 1.0 - p * jnp.exp(-x * x)


def _body(spc_ref, spr_ref, cc_ref, cr_ref, nq_ref, s2c_ref,
          wac_ref, was_ref, wqc_ref, wqs_ref,
          c0a_ref, c0q_ref, c0qr_ref, c0er_ref, c1_ref, c2_ref, c3_ref,
          cb0_ref, cb1_ref, cb2_ref, cb3_ref,
          a0a_ref, a0q_ref, a0qr_ref, a0er_ref, a1_ref, a2_ref, a3_ref,
          ab0_ref, ab1_ref, ab2_ref, ab3_ref,
          en_ref, q_ref, nbr_ref):
    pid = pl.program_id(0)

    @pl.when(pid == 0)
    def _init():
        nbr_ref[...] = jnp.zeros((T, T), jnp.float32)

    sp_c = spc_ref[0]
    sp_r = spr_ref[0]
    onehot = (sp_c == jax.lax.broadcasted_iota(jnp.int32, (T, NS), 1)
              ).astype(jnp.float32)
    sig2_c = jnp.dot(onehot, s2c_ref[...])
    sig2_r = jnp.full((1, T), SIG2[0], jnp.float32)
    for e in range(1, NS):
        sig2_r = jnp.where(sp_r == e, jnp.float32(SIG2[e]), sig2_r)

    ii = jax.lax.broadcasted_iota(jnp.int32, (NA, NA), 0)
    jj = jax.lax.broadcasted_iota(jnp.int32, (NA, NA), 1)
    offm = jnp.where(ii == jj, 0.0, 1.0).astype(jnp.float32)

    jms = []
    for g in range(G):
        sl = pl.ds(g * NA, NA)
        d2 = jnp.full((NA, NA), 1e-16, jnp.float32)
        for ax in range(3):
            col = cc_ref[0, sl, ax:ax + 1]
            row = cr_ref[0, ax:ax + 1, sl]
            dif = col - row
            d2 = d2 + dif * dif
        dist = jnp.sqrt(d2) * jnp.float32(1.0 / A0)
        nbr_ref[sl, sl] = jnp.exp(-dist) * offm
        s2 = sig2_c[g * NA:(g + 1) * NA, :] + sig2_r[:, g * NA:(g + 1) * NA]
        x = dist * jax.lax.rsqrt(2.0 * s2)
        jms.append(_erf(x) / dist * offm)

    base_c = jnp.zeros((T, 384), jnp.float32)
    for ax in range(3):
        base_c = base_c + cc_ref[0, :, ax:ax + 1] * wac_ref[ax:ax + 1, :]
    phi_aev = jnp.tanh(base_c + jnp.dot(onehot, was_ref[...]))
    aev = jnp.tanh(jnp.dot(nbr_ref[...], phi_aev))

    c1 = c1_ref[...]; c2 = c2_ref[...]; c3 = c3_ref[...]
    cb1 = cb1_ref[...]; cb2 = cb2_ref[...]; cb3 = cb3_ref[...]

    def chi_tail(pre):
        h = _celu(pre)
        h = _celu(jnp.dot(h, c1) + cb1)
        h = _celu(jnp.dot(h, c2) + cb2)
        return _softplus(jnp.dot(h, c3) + cb3)

    def equil(chi):
        qs = []
        for g in range(G):
            chi_g = chi[g * NA:(g + 1) * NA, :]
            s = jnp.sum(chi_g)
            Q = nq_ref[0, 0, g]
            k_net = 1.0 + jnp.abs(Q) / s
            k_p = jnp.where(Q > 0, k_net, 1.0)
            k_n = jnp.where(Q < 0, k_net, 1.0)
            qs.append(-k_n * chi_g + k_p * (s * jnp.float32(1.0 / NA)))
        return jnp.concatenate(qs, axis=0)

    def esp_of(q):
        es = []
        for g in range(G):
            es.append(jnp.dot(jms[g], q[g * NA:(g + 1) * NA, :]))
        return jnp.concatenate(es, axis=0)

    h_aev = jnp.dot(aev, c0a_ref[...]) + cb0_ref[...]

    chi1 = chi_tail(h_aev)
    q1 = equil(chi1)
    esp1 = esp_of(q1)

    base_q = jnp.zeros((T, 64), jnp.float32)
    for ax in range(3):
        base_q = base_q + cc_ref[0, :, ax:ax + 1] * wqc_ref[ax:ax + 1, :]
    phi_qr = jnp.tanh(base_q + jnp.dot(onehot, wqs_ref[...]))
    qraev = jnp.tanh(jnp.dot(nbr_ref[...], q1 * phi_qr))

    pre2 = (h_aev + jnp.dot(qraev, c0q_ref[...])
            + q1 * c0qr_ref[...] + esp1 * c0er_ref[...])
    chi2 = chi_tail(pre2)
    q2 = equil(chi2)
    esp2 = esp_of(q2)

    en = jnp.zeros((T, 1), jnp.float32)
    for e in range(NS):
        pre = (jnp.dot(aev, a0a_ref[e]) + jnp.dot(qraev, a0q_ref[e])
               + q2 * a0qr_ref[e] + esp2 * a0er_ref[e] + ab0_ref[e])
        h = _celu(pre)
        h = _celu(jnp.dot(h, a1_ref[e]) + ab1_ref[e])
        h = _celu(jnp.dot(h, a2_ref[e]) + ab2_ref[e])
        o = jnp.dot(h, a3_ref[e]) + ab3_ref[e]
        en = en + jnp.where(sp_c == e, o, 0.0)

    parts = []
    for g in range(G):
        sl = slice(g * NA, (g + 1) * NA)
        me = jnp.sum(en[sl, :]) + 0.5 * jnp.sum(q2[sl, :] * esp2[sl, :])
        parts.append(me.reshape(1, 1, 1))
    en_ref[...] = jnp.concatenate(parts, axis=2)
    q_ref[0] = q2


def kernel(species, coordinates, net_charge, params):
    sp_col = species.reshape(STEPS, T, 1)
    sp_row = species.reshape(STEPS, 1, T)
    cf = coordinates.reshape(STEPS, T, 3)
    coords_c = jnp.pad(cf, ((0, 0), (0, 0), (0, 5)))
    coords_r = jnp.pad(cf.transpose(0, 2, 1), ((0, 0), (0, 5), (0, 0)))
    netq = net_charge.reshape(STEPS, 1, G)
    sig2 = jnp.asarray(SIG2, jnp.float32).reshape(NS, 1)

    p = params
    wac = jnp.pad(p['W_aev'][:3], ((0, 5), (0, 0)))
    was = p['W_aev'][3:]
    wqc = jnp.pad(p['W_qr'][:3], ((0, 5), (0, 0)))
    wqs = p['W_qr'][3:]
    c0 = p['chi_W0']
    c0a, c0q = c0[:384], c0[384:448]
    c0qr, c0er = c0[448:449], c0[449:450]
    a0 = p['ani_W0']
    a0a, a0q = a0[:, :384], a0[:, 384:448]
    a0qr, a0er = a0[:, 448:449], a0[:, 449:450]

    def bs(a, smem=False):
        kw = {'memory_space': pltpu.SMEM} if smem else {}
        nd = a.ndim
        return pl.BlockSpec(a.shape, lambda i, _n=nd: (0,) * _n, **kw)

    ins = [sp_col, sp_row, coords_c, coords_r, netq, sig2,
           wac, was, wqc, wqs,
           c0a, c0q, c0qr, c0er, p['chi_W1'], p['chi_W2'], p['chi_W3'],
           p['chi_b0'].reshape(1, -1), p['chi_b1'].reshape(1, -1),
           p['chi_b2'].reshape(1, -1), p['chi_b3'].reshape(1, -1),
           a0a, a0q, a0qr, a0er, p['ani_W1'], p['ani_W2'], p['ani_W3'],
           p['ani_b0'][:, None, :], p['ani_b1'][:, None, :],
           p['ani_b2'][:, None, :], p['ani_b3'][:, None, :]]

    specs = [pl.BlockSpec((1, T, 1), lambda i: (i, 0, 0)),
             pl.BlockSpec((1, 1, T), lambda i: (i, 0, 0)),
             pl.BlockSpec((1, T, 8), lambda i: (i, 0, 0)),
             pl.BlockSpec((1, 8, T), lambda i: (i, 0, 0)),
             pl.BlockSpec((1, 1, G), lambda i: (i, 0, 0),
                          memory_space=pltpu.SMEM)]
    specs += [bs(a) for a in ins[5:]]

    out_shapes = (jax.ShapeDtypeStruct((STEPS, 1, G), jnp.float32),
                  jax.ShapeDtypeStruct((STEPS, T, 1), jnp.float32))
    out_specs = (pl.BlockSpec((1, 1, G), lambda i: (i, 0, 0)),
                 pl.BlockSpec((1, T, 1), lambda i: (i, 0, 0)))

    en, q = pl.pallas_call(
        _body,
        grid=(STEPS,),
        in_specs=specs,
        out_specs=out_specs,
        out_shape=out_shapes,
        scratch_shapes=[pltpu.VMEM((T, T), jnp.float32)],
    )(*ins)

    return species, en.reshape(NM), q.reshape(NM, NA)
